# baseline (speedup 1.0000x reference)
import functools

import jax
import jax.numpy as jnp
from jax.experimental import pallas as pl

_EPS = 1e-5


def _colsum(m):
    ones = jnp.ones((1, m.shape[0]), jnp.bfloat16)
    return jnp.dot(ones, m, preferred_element_type=jnp.float32)


def _stats_kernel(xo_ref, xc_ref, out_ref):
    j = pl.program_id(0)
    xo = xo_ref[...].astype(jnp.bfloat16)
    xc = xc_ref[...].astype(jnp.bfloat16)
    rows = [
        _colsum(xo),
        _colsum(xo * xo),
        _colsum(xc),
        _colsum(xc * xc),
        _colsum(xo * xc),
    ]
    block = jnp.concatenate(
        rows + [jnp.zeros((3, xo.shape[1]), jnp.float32)], axis=0)

    @pl.when(j == 0)
    def _():
        out_ref[...] = block

    @pl.when(j > 0)
    def _():
        out_ref[...] += block


def _hstats_kernel(xo_ref, xc_ref, wc_ref, cc_ref, wo_ref, co_ref, wr_ref, cr_ref,
                   out_ref):
    j = pl.program_id(0)
    xo = xo_ref[...].astype(jnp.bfloat16)
    xc = xc_ref[...].astype(jnp.bfloat16)
    xr = xo + xc
    rows = []
    for x, w_ref, c_ref in ((xc, wc_ref, cc_ref), (xo, wo_ref, co_ref),
                            (xr, wr_ref, cr_ref)):
        h = jnp.maximum(
            jnp.dot(x, w_ref[...], preferred_element_type=jnp.float32)
            + c_ref[...], 0.0).astype(jnp.bfloat16)
        rows.append(_colsum(h))
        rows.append(_colsum(h * h))
    block = jnp.concatenate(
        rows + [jnp.zeros((2, xo.shape[1]), jnp.float32)], axis=0)

    @pl.when(j == 0)
    def _():
        out_ref[...] = block

    @pl.when(j > 0)
    def _():
        out_ref[...] += block


def _final_kernel(xo_ref, xc_ref,
                  wc1_ref, cc1_ref, wo1_ref, co1_ref, wr1_ref, cr1_ref,
                  wc2_ref, cc2_ref, wo2_ref, co2_ref, wr2_ref, cr2_ref,
                  oc_ref, oo_ref, or_ref):
    xo = xo_ref[...].astype(jnp.bfloat16)
    xc = xc_ref[...].astype(jnp.bfloat16)
    xr = xo + xc

    def head(x, w1_ref, c1_ref, w2_ref, c2_ref):
        h = jnp.maximum(
            jnp.dot(x, w1_ref[...], preferred_element_type=jnp.float32)
            + c1_ref[...], 0.0).astype(jnp.bfloat16)
        return (jnp.dot(h, w2_ref[...], preferred_element_type=jnp.float32)
                + c2_ref[...])

    def log_softmax(z):
        m = jnp.max(z, axis=-1, keepdims=True)
        s = z - m
        return s - jnp.log(jnp.sum(jnp.exp(s), axis=-1, keepdims=True))

    oc_ref[...] = log_softmax(head(xc, wc1_ref, cc1_ref, wc2_ref, cc2_ref))
    oo_ref[...] = head(xo, wo1_ref, co1_ref, wo2_ref, co2_ref)
    or_ref[...] = log_softmax(head(xr, wr1_ref, cr1_ref, wr2_ref, cr2_ref))


def _row_spec(r, h):
    return pl.BlockSpec((r, h), lambda j: (j, 0))


def _rep_spec(shape):
    return pl.BlockSpec(shape, lambda j: tuple(0 for _ in shape))


def _fold1(m, v, g, b, W, c):
    a = g / jnp.sqrt(v + _EPS)
    wp = (a[:, None] * W).astype(jnp.bfloat16)
    return wp, ((b - a * m)[None, :] @ W) + c[None, :]


@functools.partial(jax.jit, static_argnames=())
def kernel(xo, xc,
           ctx_g1, ctx_b1, ctx_W1, ctx_c1, ctx_g2, ctx_b2, ctx_W2, ctx_c2,
           obj_g1, obj_b1, obj_W1, obj_c1, obj_g2, obj_b2, obj_W2, obj_c2,
           rnd_g1, rnd_b1, rnd_W1, rnd_c1, rnd_g2, rnd_b2, rnd_W2, rnd_c2):
    B, H = xo.shape
    O = ctx_W2.shape[1]
    R = 2000 if B % 2000 == 0 else (1000 if B % 1000 == 0 else B)
    nb = B // R

    stats = pl.pallas_call(
        _stats_kernel,
        grid=(nb,),
        in_specs=[_row_spec(R, H), _row_spec(R, H)],
        out_specs=_rep_spec((8, H)),
        out_shape=jax.ShapeDtypeStruct((8, H), jnp.float32),
    )(xo, xc)

    inv_b = 1.0 / B
    m_xo = stats[0] * inv_b
    v_xo = stats[1] * inv_b - m_xo * m_xo
    m_xc = stats[2] * inv_b
    v_xc = stats[3] * inv_b - m_xc * m_xc
    m_xr = m_xo + m_xc
    v_xr = (stats[1] + stats[3] + 2.0 * stats[4]) * inv_b - m_xr * m_xr

    wc1, cc1 = _fold1(m_xc, v_xc, ctx_g1, ctx_b1, ctx_W1, ctx_c1)
    wo1, co1 = _fold1(m_xo, v_xo, obj_g1, obj_b1, obj_W1, obj_c1)
    wr1, cr1 = _fold1(m_xr, v_xr, rnd_g1, rnd_b1, rnd_W1, rnd_c1)

    hstats = pl.pallas_call(
        _hstats_kernel,
        grid=(nb,),
        in_specs=[_row_spec(R, H), _row_spec(R, H),
                  _rep_spec((H, H)), _rep_spec((1, H)),
                  _rep_spec((H, H)), _rep_spec((1, H)),
                  _rep_spec((H, H)), _rep_spec((1, H))],
        out_specs=_rep_spec((8, H)),
        out_shape=jax.ShapeDtypeStruct((8, H), jnp.float32),
    )(xo, xc, wc1, cc1, wo1, co1, wr1, cr1)

    m_hc = hstats[0] * inv_b
    v_hc = hstats[1] * inv_b - m_hc * m_hc
    m_ho = hstats[2] * inv_b
    v_ho = hstats[3] * inv_b - m_ho * m_ho
    m_hr = hstats[4] * inv_b
    v_hr = hstats[5] * inv_b - m_hr * m_hr

    wc2, cc2 = _fold1(m_hc, v_hc, ctx_g2, ctx_b2, ctx_W2, ctx_c2)
    wo2, co2 = _fold1(m_ho, v_ho, obj_g2, obj_b2, obj_W2, obj_c2)
    wr2, cr2 = _fold1(m_hr, v_hr, rnd_g2, rnd_b2, rnd_W2, rnd_c2)

    outs = pl.pallas_call(
        _final_kernel,
        grid=(nb,),
        in_specs=[_row_spec(R, H), _row_spec(R, H),
                  _rep_spec((H, H)), _rep_spec((1, H)),
                  _rep_spec((H, H)), _rep_spec((1, H)),
                  _rep_spec((H, H)), _rep_spec((1, H)),
                  _rep_spec((H, O)), _rep_spec((1, O)),
                  _rep_spec((H, O)), _rep_spec((1, O)),
                  _rep_spec((H, O)), _rep_spec((1, O))],
        out_specs=[_row_spec(R, O), _row_spec(R, O), _row_spec(R, O)],
        out_shape=[jax.ShapeDtypeStruct((B, O), jnp.float32)] * 3,
    )(xo, xc, wc1, cc1, wo1, co1, wr1, cr1,
      wc2, cc2, wo2, co2, wr2, cr2)

    return tuple(outs)


# speedup vs baseline: 1.1672x; 1.1672x over previous
import functools

import jax
import jax.numpy as jnp
from jax.experimental import pallas as pl
from jax.experimental.pallas import tpu as pltpu

_EPS = 1e-5


def _csum(x):
    return jnp.sum(x, axis=0, keepdims=True)


def _stage_kernel(xo_ref, xc_ref, stats_ref, xob_ref, xcb_ref):
    j = pl.program_id(0)
    xo = xo_ref[...]
    xc = xc_ref[...]
    xob_ref[...] = xo.astype(jnp.bfloat16)
    xcb_ref[...] = xc.astype(jnp.bfloat16)
    block = jnp.concatenate([
        _csum(xo), _csum(xo * xo), _csum(xc), _csum(xc * xc),
        _csum(xo * xc), jnp.zeros((3, xo.shape[1]), jnp.float32)
    ], axis=0)

    @pl.when(j == 0)
    def _():
        stats_ref[...] = block

    @pl.when(j > 0)
    def _():
        stats_ref[...] += block


def _main_kernel(nb, inv_b,
                 xob_ref, xcb_ref, stats_ref,
                 cg1_ref, cb1_ref, cw1_ref, cc1_ref,
                 og1_ref, ob1_ref, ow1_ref, oc1_ref,
                 rg1_ref, rb1_ref, rw1_ref, rc1_ref,
                 cg2_ref, cb2_ref, cw2_ref, cc2_ref,
                 og2_ref, ob2_ref, ow2_ref, oc2_ref,
                 rg2_ref, rb2_ref, rw2_ref, rc2_ref,
                 out_c_ref, out_o_ref, out_r_ref,
                 hstats_ref, scale_ref, bias_ref):
    p = pl.program_id(0)
    j = pl.program_id(1)

    def fold(row, m, v, g_ref, b_ref, w_ref, c_ref):
        a = g_ref[...] * jax.lax.rsqrt(v + _EPS)
        scale_ref[row:row + 1, :] = a.astype(jnp.bfloat16)
        k = (b_ref[...] - a * m).astype(jnp.bfloat16)
        bias_ref[row:row + 1, :] = (
            jnp.dot(k, w_ref[...], preferred_element_type=jnp.float32)
            + c_ref[...])

    @pl.when((p == 0) & (j == 0))
    def _():
        s = stats_ref[...]
        m_xo = s[0:1] * inv_b
        v_xo = s[1:2] * inv_b - m_xo * m_xo
        m_xc = s[2:3] * inv_b
        v_xc = s[3:4] * inv_b - m_xc * m_xc
        m_xr = m_xo + m_xc
        v_xr = (s[1:2] + s[3:4] + 2.0 * s[4:5]) * inv_b - m_xr * m_xr
        fold(0, m_xc, v_xc, cg1_ref, cb1_ref, cw1_ref, cc1_ref)
        fold(1, m_xo, v_xo, og1_ref, ob1_ref, ow1_ref, oc1_ref)
        fold(2, m_xr, v_xr, rg1_ref, rb1_ref, rw1_ref, rc1_ref)

    @pl.when((p == 1) & (j == 0))
    def _():
        hs = hstats_ref[...]
        for row, (g_ref, b_ref, w_ref, c_ref) in enumerate(
                ((cg2_ref, cb2_ref, cw2_ref, cc2_ref),
                 (og2_ref, ob2_ref, ow2_ref, oc2_ref),
                 (rg2_ref, rb2_ref, rw2_ref, rc2_ref))):
            m = hs[2 * row:2 * row + 1] * inv_b
            v = hs[2 * row + 1:2 * row + 2] * inv_b - m * m
            fold(3 + row, m, v, g_ref, b_ref, w_ref, c_ref)

    xo = xob_ref[...]
    xc = xcb_ref[...]
    xr = xo + xc

    def hidden(row, x, w1_ref):
        xa = x * scale_ref[row:row + 1, :]
        return jnp.maximum(
            jnp.dot(xa, w1_ref[...], preferred_element_type=jnp.float32)
            + bias_ref[row:row + 1, :], 0.0)

    @pl.when(p == 0)
    def _():
        rows = []
        for row, (x, w1_ref) in enumerate(((xc, cw1_ref), (xo, ow1_ref),
                                           (xr, rw1_ref))):
            h = hidden(row, x, w1_ref)
            rows.append(_csum(h))
            rows.append(_csum(h * h))
        block = jnp.concatenate(
            rows + [jnp.zeros((2, xo.shape[1]), jnp.float32)], axis=0)

        @pl.when(j == 0)
        def _():
            hstats_ref[...] = block

        @pl.when(j > 0)
        def _():
            hstats_ref[...] += block

    @pl.when(p == 1)
    def _():
        def head(row, x, w1_ref, w2_ref):
            hb = hidden(row, x, w1_ref).astype(jnp.bfloat16)
            hb = hb * scale_ref[3 + row:4 + row, :]
            return (jnp.dot(hb, w2_ref[...], preferred_element_type=jnp.float32)
                    + bias_ref[3 + row:4 + row, :])

        def log_softmax(z):
            m = jnp.max(z, axis=-1, keepdims=True)
            s = z - m
            return s - jnp.log(jnp.sum(jnp.exp(s), axis=-1, keepdims=True))

        out_c_ref[...] = log_softmax(head(0, xc, cw1_ref, cw2_ref))
        out_o_ref[...] = head(1, xo, ow1_ref, ow2_ref)
        out_r_ref[...] = log_softmax(head(2, xr, rw1_ref, rw2_ref))


def _row1(r, h):
    return pl.BlockSpec((r, h), lambda j: (j, 0))


def _vec2(h):
    return pl.BlockSpec((1, h), lambda p, j: (0, 0))


def _mat2(h, o):
    return pl.BlockSpec((h, o), lambda p, j: (0, 0))


@functools.partial(jax.jit, static_argnames=())
def kernel(xo, xc,
           ctx_g1, ctx_b1, ctx_W1, ctx_c1, ctx_g2, ctx_b2, ctx_W2, ctx_c2,
           obj_g1, obj_b1, obj_W1, obj_c1, obj_g2, obj_b2, obj_W2, obj_c2,
           rnd_g1, rnd_b1, rnd_W1, rnd_c1, rnd_g2, rnd_b2, rnd_W2, rnd_c2):
    B, H = xo.shape
    O = ctx_W2.shape[1]
    R = 2000 if B % 2000 == 0 else (1000 if B % 1000 == 0 else B)
    nb = B // R

    stats, xob, xcb = pl.pallas_call(
        _stage_kernel,
        grid=(nb,),
        in_specs=[_row1(R, H), _row1(R, H)],
        out_specs=[pl.BlockSpec((8, H), lambda j: (0, 0)),
                   _row1(R, H), _row1(R, H)],
        out_shape=[jax.ShapeDtypeStruct((8, H), jnp.float32),
                   jax.ShapeDtypeStruct((B, H), jnp.bfloat16),
                   jax.ShapeDtypeStruct((B, H), jnp.bfloat16)],
    )(xo, xc)

    w1s = [w.astype(jnp.bfloat16) for w in (ctx_W1, obj_W1, rnd_W1)]
    w2s = [w.astype(jnp.bfloat16) for w in (ctx_W2, obj_W2, rnd_W2)]
    vecs = {k: v.reshape(1, H) for k, v in dict(
        cg1=ctx_g1, cb1=ctx_b1, cc1=ctx_c1, og1=obj_g1, ob1=obj_b1,
        oc1=obj_c1, rg1=rnd_g1, rb1=rnd_b1, rc1=rnd_c1,
        cg2=ctx_g2, cb2=ctx_b2, cc2=ctx_c2, og2=obj_g2, ob2=obj_b2,
        oc2=obj_c2, rg2=rnd_g2, rb2=rnd_b2, rc2=rnd_c2).items()}

    row_in = pl.BlockSpec((R, H), lambda p, j: (j, 0))
    row_out = pl.BlockSpec((R, O), lambda p, j: (p * j, 0))

    outs = pl.pallas_call(
        functools.partial(_main_kernel, nb, 1.0 / B),
        grid=(2, nb),
        in_specs=[row_in, row_in, pl.BlockSpec((8, H), lambda p, j: (0, 0)),
                  _vec2(H), _vec2(H), _mat2(H, H), _vec2(H),
                  _vec2(H), _vec2(H), _mat2(H, H), _vec2(H),
                  _vec2(H), _vec2(H), _mat2(H, H), _vec2(H),
                  _vec2(H), _vec2(H), _mat2(H, O), _vec2(O),
                  _vec2(H), _vec2(H), _mat2(H, O), _vec2(O),
                  _vec2(H), _vec2(H), _mat2(H, O), _vec2(O)],
        out_specs=[row_out, row_out, row_out],
        out_shape=[jax.ShapeDtypeStruct((B, O), jnp.float32)] * 3,
        scratch_shapes=[pltpu.VMEM((8, H), jnp.float32),
                        pltpu.VMEM((8, H), jnp.bfloat16),
                        pltpu.VMEM((8, H), jnp.float32)],
    )(xob, xcb, stats,
      vecs["cg1"], vecs["cb1"], w1s[0], vecs["cc1"],
      vecs["og1"], vecs["ob1"], w1s[1], vecs["oc1"],
      vecs["rg1"], vecs["rb1"], w1s[2], vecs["rc1"],
      vecs["cg2"], vecs["cb2"], w2s[0], vecs["cc2"],
      vecs["og2"], vecs["ob2"], w2s[1], vecs["oc2"],
      vecs["rg2"], vecs["rb2"], w2s[2], vecs["rc2"])

    return tuple(outs)


# speedup vs baseline: 1.2122x; 1.0385x over previous
import functools

import jax
import jax.numpy as jnp
from jax.experimental import pallas as pl
from jax.experimental.pallas import tpu as pltpu

_EPS = 1e-5


def _csum(x):
    return jnp.sum(x, axis=0, keepdims=True)


def _stage_kernel(xo_ref, xc_ref, stats_ref, xob_ref, xcb_ref):
    j = pl.program_id(0)
    xo = xo_ref[...]
    xc = xc_ref[...]
    xob_ref[...] = xo.astype(jnp.bfloat16)
    xcb_ref[...] = xc.astype(jnp.bfloat16)
    block = jnp.concatenate([
        _csum(xo), _csum(xo * xo), _csum(xc), _csum(xc * xc),
        _csum(xo * xc), jnp.zeros((3, xo.shape[1]), jnp.float32)
    ], axis=0)

    @pl.when(j == 0)
    def _():
        stats_ref[...] = block

    @pl.when(j > 0)
    def _():
        stats_ref[...] += block


def _main_kernel(inv_b,
                 xob_ref, xcb_ref, stats_ref,
                 cg1_ref, cb1_ref, cw1_ref, cc1_ref,
                 og1_ref, ob1_ref, ow1_ref, oc1_ref,
                 rg1_ref, rb1_ref, rw1_ref, rc1_ref,
                 cg2_ref, cb2_ref, cw2_ref, cc2_ref,
                 og2_ref, ob2_ref, ow2_ref, oc2_ref,
                 rg2_ref, rb2_ref, rw2_ref, rc2_ref,
                 out_c_ref, out_o_ref, out_r_ref,
                 w1big_ref, b1big_ref, w2c_ref, w2o_ref, w2r_ref, b2_ref,
                 hstats_ref):
    p = pl.program_id(0)
    j = pl.program_id(1)
    H = cw1_ref.shape[0]

    def fold(m, v, g_ref, b_ref, w_ref, c_ref):
        a = g_ref[...] * jax.lax.rsqrt(v + _EPS)
        ws = (jnp.transpose(a) * w_ref[...]).astype(jnp.bfloat16)
        bias = (jnp.dot(b_ref[...] - a * m, w_ref[...],
                        preferred_element_type=jnp.float32) + c_ref[...])
        return ws, bias

    @pl.when((p == 0) & (j == 0))
    def _():
        s = stats_ref[...]
        m_xo = s[0:1] * inv_b
        v_xo = s[1:2] * inv_b - m_xo * m_xo
        m_xc = s[2:3] * inv_b
        v_xc = s[3:4] * inv_b - m_xc * m_xc
        m_xr = m_xo + m_xc
        v_xr = (s[1:2] + s[3:4] + 2.0 * s[4:5]) * inv_b - m_xr * m_xr
        wc, bc = fold(m_xc, v_xc, cg1_ref, cb1_ref, cw1_ref, cc1_ref)
        wo, bo = fold(m_xo, v_xo, og1_ref, ob1_ref, ow1_ref, oc1_ref)
        wr, br = fold(m_xr, v_xr, rg1_ref, rb1_ref, rw1_ref, rc1_ref)
        z = jnp.zeros((H, H), jnp.bfloat16)
        w1big_ref[...] = jnp.concatenate([
            jnp.concatenate([wc, z, wr], axis=1),
            jnp.concatenate([z, wo, wr], axis=1)], axis=0)
        b1big_ref[...] = jnp.concatenate([bc, bo, br], axis=1)

    @pl.when((p == 1) & (j == 0))
    def _():
        hs = hstats_ref[...]
        for k, (g_ref, b_ref, w_ref, c_ref, w2_ref) in enumerate(
                ((cg2_ref, cb2_ref, cw2_ref, cc2_ref, w2c_ref),
                 (og2_ref, ob2_ref, ow2_ref, oc2_ref, w2o_ref),
                 (rg2_ref, rb2_ref, rw2_ref, rc2_ref, w2r_ref))):
            m = hs[0:1, k * H:(k + 1) * H] * inv_b
            v = hs[1:2, k * H:(k + 1) * H] * inv_b - m * m
            ws, bias = fold(m, v, g_ref, b_ref, w_ref, c_ref)
            w2_ref[...] = ws
            b2_ref[k:k + 1, :] = bias

    xbig = jnp.concatenate([xcb_ref[...], xob_ref[...]], axis=1)
    h = jnp.maximum(
        jnp.dot(xbig, w1big_ref[...], preferred_element_type=jnp.float32)
        + b1big_ref[...], 0.0)

    @pl.when(p == 0)
    def _():
        block = jnp.concatenate(
            [_csum(h), _csum(h * h),
             jnp.zeros((6, h.shape[1]), jnp.float32)], axis=0)

        @pl.when(j == 0)
        def _():
            hstats_ref[...] = block

        @pl.when(j > 0)
        def _():
            hstats_ref[...] += block

    @pl.when(p == 1)
    def _():
        hb = h.astype(jnp.bfloat16)

        def head(k, w2_ref):
            return (jnp.dot(hb[:, k * H:(k + 1) * H], w2_ref[...],
                            preferred_element_type=jnp.float32)
                    + b2_ref[k:k + 1, :])

        def log_softmax(z):
            m = jnp.max(z, axis=-1, keepdims=True)
            s = z - m
            return s - jnp.log(jnp.sum(jnp.exp(s), axis=-1, keepdims=True))

        out_c_ref[...] = log_softmax(head(0, w2c_ref))
        out_o_ref[...] = head(1, w2o_ref)
        out_r_ref[...] = log_softmax(head(2, w2r_ref))


def _row1(r, h):
    return pl.BlockSpec((r, h), lambda j: (j, 0))


def _vec2(h):
    return pl.BlockSpec((1, h), lambda p, j: (0, 0))


def _mat2(h, o):
    return pl.BlockSpec((h, o), lambda p, j: (0, 0))


@functools.partial(jax.jit, static_argnames=())
def kernel(xo, xc,
           ctx_g1, ctx_b1, ctx_W1, ctx_c1, ctx_g2, ctx_b2, ctx_W2, ctx_c2,
           obj_g1, obj_b1, obj_W1, obj_c1, obj_g2, obj_b2, obj_W2, obj_c2,
           rnd_g1, rnd_b1, rnd_W1, rnd_c1, rnd_g2, rnd_b2, rnd_W2, rnd_c2):
    B, H = xo.shape
    O = ctx_W2.shape[1]
    R = 2000 if B % 2000 == 0 else (1000 if B % 1000 == 0 else B)
    nb = B // R

    stats, xob, xcb = pl.pallas_call(
        _stage_kernel,
        grid=(nb,),
        in_specs=[_row1(R, H), _row1(R, H)],
        out_specs=[pl.BlockSpec((8, H), lambda j: (0, 0)),
                   _row1(R, H), _row1(R, H)],
        out_shape=[jax.ShapeDtypeStruct((8, H), jnp.float32),
                   jax.ShapeDtypeStruct((B, H), jnp.bfloat16),
                   jax.ShapeDtypeStruct((B, H), jnp.bfloat16)],
    )(xo, xc)

    vecs = {k: v.reshape(1, H) for k, v in dict(
        cg1=ctx_g1, cb1=ctx_b1, cc1=ctx_c1, og1=obj_g1, ob1=obj_b1,
        oc1=obj_c1, rg1=rnd_g1, rb1=rnd_b1, rc1=rnd_c1,
        cg2=ctx_g2, cb2=ctx_b2, cc2=ctx_c2, og2=obj_g2, ob2=obj_b2,
        oc2=obj_c2, rg2=rnd_g2, rb2=rnd_b2, rc2=rnd_c2).items()}

    row_in = pl.BlockSpec((R, H), lambda p, j: (j, 0))
    row_out = pl.BlockSpec((R, O), lambda p, j: (p * j, 0))

    outs = pl.pallas_call(
        functools.partial(_main_kernel, 1.0 / B),
        grid=(2, nb),
        in_specs=[row_in, row_in, pl.BlockSpec((8, H), lambda p, j: (0, 0)),
                  _vec2(H), _vec2(H), _mat2(H, H), _vec2(H),
                  _vec2(H), _vec2(H), _mat2(H, H), _vec2(H),
                  _vec2(H), _vec2(H), _mat2(H, H), _vec2(H),
                  _vec2(H), _vec2(H), _mat2(H, O), _vec2(O),
                  _vec2(H), _vec2(H), _mat2(H, O), _vec2(O),
                  _vec2(H), _vec2(H), _mat2(H, O), _vec2(O)],
        out_specs=[row_out, row_out, row_out],
        out_shape=[jax.ShapeDtypeStruct((B, O), jnp.float32)] * 3,
        scratch_shapes=[pltpu.VMEM((2 * H, 3 * H), jnp.bfloat16),
                        pltpu.VMEM((1, 3 * H), jnp.float32),
                        pltpu.VMEM((H, O), jnp.bfloat16),
                        pltpu.VMEM((H, O), jnp.bfloat16),
                        pltpu.VMEM((H, O), jnp.bfloat16),
                        pltpu.VMEM((8, O), jnp.float32),
                        pltpu.VMEM((8, 3 * H), jnp.float32)],
    )(xob, xcb, stats,
      vecs["cg1"], vecs["cb1"], ctx_W1, vecs["cc1"],
      vecs["og1"], vecs["ob1"], obj_W1, vecs["oc1"],
      vecs["rg1"], vecs["rb1"], rnd_W1, vecs["rc1"],
      vecs["cg2"], vecs["cb2"], ctx_W2, vecs["cc2"],
      vecs["og2"], vecs["ob2"], obj_W2, vecs["oc2"],
      vecs["rg2"], vecs["rb2"], rnd_W2, vecs["rc2"])

    return tuple(outs)
